# baseline (speedup 1.0000x reference)
import dataclasses
import functools

import jax
import jax.numpy as jnp
from jax import lax
from jax.experimental import pallas as pl
from jax.experimental.pallas import tpu as pltpu
from jax.experimental.pallas import tpu_sc as plsc


def _body(cur_smem, nxt_smem, xv_ref, emb_ref, dummy_ref, out_ref, buf_ref,
          sem, *, L):
    t = pl.program_id(0)
    nt = pl.num_programs(0)
    TB = out_ref.shape[0]

    def issue(idx_smem, slot):
        def issue_rows(b, carry):
            for l in range(L):
                idx = idx_smem[b, l]
                pltpu.make_async_copy(
                    emb_ref.at[idx], buf_ref.at[slot, b, l], sem.at[slot]
                ).start()
            return carry

        jax.lax.fori_loop(0, TB, issue_rows, 0)

    def drain(slot):
        def drain_rows(b, carry):
            for l in range(L):
                pltpu.make_async_copy(
                    dummy_ref.at[l], buf_ref.at[slot, b, l], sem.at[slot]
                ).wait()
            return carry

        jax.lax.fori_loop(0, TB, drain_rows, 0)

    parity = jax.lax.rem(t, 2)

    @pl.when(t == 0)
    def _():
        issue(cur_smem, 0)

    @pl.when(jnp.logical_and(t + 1 < nt, parity == 0))
    def _():
        issue(nxt_smem, 1)

    @pl.when(jnp.logical_and(t + 1 < nt, parity == 1))
    def _():
        issue(nxt_smem, 0)

    @pl.when(parity == 0)
    def _():
        drain(0)

    @pl.when(parity == 1)
    def _():
        drain(1)

    mask = (xv_ref[...] != 0).astype(jnp.float32)

    @pl.when(parity == 0)
    def _():
        out_ref[...] = jnp.tanh(jnp.sum(buf_ref[0] * mask, axis=1))

    @pl.when(parity == 1)
    def _():
        out_ref[...] = jnp.tanh(jnp.sum(buf_ref[1] * mask, axis=1))


def _phedvec(x, embeddings, tb, interpret=False):
    B, L = x.shape
    _, D = embeddings.shape
    nt = B // tb
    grid_spec = pltpu.PrefetchScalarGridSpec(
        num_scalar_prefetch=0,
        grid=(nt,),
        in_specs=[
            pl.BlockSpec((tb, L), lambda t: (t, 0), memory_space=pltpu.SMEM),
            pl.BlockSpec(
                (tb, L),
                lambda t: (jnp.minimum(t + 1, nt - 1), 0),
                memory_space=pltpu.SMEM,
            ),
            pl.BlockSpec((tb, L, 1), lambda t: (t, 0, 0)),
            pl.BlockSpec(memory_space=pltpu.HBM),
            pl.BlockSpec(memory_space=pltpu.HBM),
        ],
        out_specs=pl.BlockSpec((tb, D), lambda t: (t, 0)),
        scratch_shapes=[
            pltpu.VMEM((2, tb, L, D), jnp.float32),
            pltpu.SemaphoreType.DMA((2,)),
        ],
    )
    return pl.pallas_call(
        functools.partial(_body, L=L),
        grid_spec=grid_spec,
        out_shape=jax.ShapeDtypeStruct((B, D), jnp.float32),
        compiler_params=pltpu.CompilerParams(
            dimension_semantics=("arbitrary",),
        ),
        interpret=interpret,
    )(x, x, x.reshape(B, L, 1), embeddings,
      jnp.zeros((L, D), jnp.float32))


_NC = 2
_NS = 16
_NW = _NC * _NS
_CHUNK = 16


def _sc_sums(x, embeddings):
    B, L = x.shape
    _, D = embeddings.shape
    PW = B // _NW
    NF = D // _CHUNK if D % _CHUNK == 0 else (D - D % _CHUNK) // _CHUNK
    mesh = plsc.VectorSubcoreMesh(core_axis_name="c", subcore_axis_name="s")
    sc_params = pltpu.CompilerParams()
    if "use_tc_tiling_on_sc" in pltpu.CompilerParams.__dataclass_fields__:
        sc_params = dataclasses.replace(sc_params, use_tc_tiling_on_sc=False)

    @functools.partial(
        pl.kernel,
        out_type=jax.ShapeDtypeStruct((B, D), jnp.float32),
        mesh=mesh,
        compiler_params=sc_params,
        scratch_types=[
            pltpu.VMEM((PW, L), jnp.int32),
            pltpu.VMEM((2, L, D), jnp.float32),
            pltpu.VMEM((2, 2, D), jnp.float32),
            pltpu.SemaphoreType.DMA((2,)),
            pltpu.SemaphoreType.DMA((2,)),
        ],
    )
    def sums_kernel(x_hbm, table_hbm, out_hbm, idx_v, rows_v, ostage, gsem,
                    osem):
        wid = lax.axis_index("s") * _NC + lax.axis_index("c")
        base = wid * PW

        pltpu.sync_copy(x_hbm.at[pl.ds(base, PW)], idx_v)

        def start_gather(j, slot):
            pltpu.make_async_copy(
                table_hbm.at[idx_v.at[j]], rows_v.at[slot], gsem.at[slot]
            ).start()

        def wait_gather(j, slot):
            pltpu.make_async_copy(
                table_hbm.at[idx_v.at[j]], rows_v.at[slot], gsem.at[slot]
            ).wait()

        def accum(rslot, oslot, orow):
            src = rows_v.at[rslot]

            @pl.loop(0, NF)
            def _(c):
                off = c * _CHUNK
                acc = jnp.zeros((_CHUNK,), jnp.float32)
                for r in range(L):
                    acc = acc + src[r, pl.ds(off, _CHUNK)]
                ostage[oslot, orow, pl.ds(off, _CHUNK)] = acc

            if D % _CHUNK:
                acc = jnp.zeros((_CHUNK,), jnp.float32)
                for r in range(L):
                    acc = acc + src[r, pl.ds(D - _CHUNK, _CHUNK)]
                ostage[oslot, orow, pl.ds(D - _CHUNK, _CHUNK)] = acc

        def start_out(oslot, j):
            pltpu.make_async_copy(
                ostage.at[oslot], out_hbm.at[pl.ds(base + j, 2)],
                osem.at[oslot],
            ).start()

        def wait_out(oslot, j):
            pltpu.make_async_copy(
                ostage.at[oslot], out_hbm.at[pl.ds(base + j, 2)],
                osem.at[oslot],
            ).wait()

        start_gather(0, 0)
        start_gather(1, 1)

        @pl.loop(0, PW, step=4)
        def _(j):
            @pl.when(j >= 4)
            def _():
                wait_out(0, j - 4)

            wait_gather(j, 0)
            accum(0, 0, 0)
            start_gather(j + 2, 0)
            wait_gather(j + 1, 1)
            accum(1, 0, 1)
            start_gather(j + 3, 1)
            start_out(0, j)

            @pl.when(j >= 4)
            def _():
                wait_out(1, j - 2)

            wait_gather(j + 2, 0)
            accum(0, 1, 0)

            @pl.when(j + 4 < PW)
            def _():
                start_gather(j + 4, 0)

            wait_gather(j + 3, 1)
            accum(1, 1, 1)

            @pl.when(j + 5 < PW)
            def _():
                start_gather(j + 5, 1)

            start_out(1, j + 2)

        wait_out(0, PW - 4)
        wait_out(1, PW - 2)

    return sums_kernel(x, embeddings)


def _fix_body(acc_ref, xv_ref, e0_ref, out_ref):
    n0 = jnp.sum((xv_ref[...] == 0).astype(jnp.float32), axis=1)
    out_ref[...] = jnp.tanh(acc_ref[...] - n0 * e0_ref[...])


def _tanh_fix(sums, x3, e0, tb):
    B, D = sums.shape
    nt = B // tb
    return pl.pallas_call(
        _fix_body,
        grid=(nt,),
        in_specs=[
            pl.BlockSpec((tb, D), lambda t: (t, 0)),
            pl.BlockSpec((tb, x3.shape[1], 1), lambda t: (t, 0, 0)),
            pl.BlockSpec((1, D), lambda t: (0, 0)),
        ],
        out_specs=pl.BlockSpec((tb, D), lambda t: (t, 0)),
        out_shape=jax.ShapeDtypeStruct((B, D), jnp.float32),
        compiler_params=pltpu.CompilerParams(
            dimension_semantics=("arbitrary",),
        ),
    )(sums, x3, e0)


def kernel(x, embeddings):
    xi = x.astype(jnp.int32)
    B, L = xi.shape
    sums = _sc_sums(xi, embeddings)
    e0 = lax.slice(embeddings, (0, 0), (1, embeddings.shape[1]))
    return _tanh_fix(sums, xi.reshape(B, L, 1), e0, tb=256)
